# baseline (speedup 1.0000x reference)
import functools

import jax
import jax.numpy as jnp
from jax import lax
from jax.experimental import pallas as pl
from jax.experimental.pallas import tpu as pltpu
from jax.experimental.pallas import tpu_sc as plsc

N = 10000
T = 12
H = 128
OUT = 32
E = 320000

NC = 2
NS = 16
CH = 128
F0 = 48
F1 = 120
FMAX = 120
PH = 24
NPH = FMAX // PH
NCHUNK = F0 + F1
EPAD = NS * NCHUNK * CH
E0 = NS * F0 * CH
NBUF = 2
NPAD = 10240
ROWS_PER_TILE_Z = NPAD // NS

BN_GRU = 400
BN_D = 1000


def _gru_body(x_ref, wih0, whh0, bih0, bhh0, wih1, whh1, bih1, bhh1, wg0,
              h_ref, m_ref, h1_scr, h2_scr):
    t = pl.program_id(1)

    @pl.when(t == 0)
    def _():
        h1_scr[...] = jnp.zeros_like(h1_scr)
        h2_scr[...] = jnp.zeros_like(h2_scr)

    xt = x_ref[0]

    def step(xin, h, wih, whh, bih, bhh):
        gi = jnp.dot(xin, wih[...], preferred_element_type=jnp.float32) + bih[...]
        gh = jnp.dot(h, whh[...], preferred_element_type=jnp.float32) + bhh[...]
        r = jax.nn.sigmoid(gi[:, :H] + gh[:, :H])
        z = jax.nn.sigmoid(gi[:, H:2 * H] + gh[:, H:2 * H])
        n = jnp.tanh(gi[:, 2 * H:] + r * gh[:, 2 * H:])
        return (1.0 - z) * n + z * h

    h1 = step(xt, h1_scr[...], wih0, whh0, bih0, bhh0)
    h2 = step(h1, h2_scr[...], wih1, whh1, bih1, bhh1)
    h1_scr[...] = h1
    h2_scr[...] = h2

    @pl.when(t == T - 1)
    def _():
        h_ref[...] = h2
        m_ref[...] = jnp.dot(h2, wg0[...], preferred_element_type=jnp.float32)


def _run_gru_tc(xt, Wih0T, Whh0T, bih0, bhh0, Wih1T, Whh1T, bih1, bhh1, Wg0T):
    nb = pl.cdiv(N, BN_GRU)
    full = lambda i, t: (0, 0)
    return pl.pallas_call(
        _gru_body,
        grid=(nb, T),
        in_specs=[
            pl.BlockSpec((1, BN_GRU, H), lambda i, t: (t, i, 0)),
            pl.BlockSpec((H, 3 * H), full),
            pl.BlockSpec((H, 3 * H), full),
            pl.BlockSpec((1, 3 * H), full),
            pl.BlockSpec((1, 3 * H), full),
            pl.BlockSpec((H, 3 * H), full),
            pl.BlockSpec((H, 3 * H), full),
            pl.BlockSpec((1, 3 * H), full),
            pl.BlockSpec((1, 3 * H), full),
            pl.BlockSpec((H, H), full),
        ],
        out_specs=[
            pl.BlockSpec((BN_GRU, H), lambda i, t: (i, 0)),
            pl.BlockSpec((BN_GRU, H), lambda i, t: (i, 0)),
        ],
        out_shape=[
            jax.ShapeDtypeStruct((N, H), jnp.float32),
            jax.ShapeDtypeStruct((N, H), jnp.float32),
        ],
        scratch_shapes=[
            pltpu.VMEM((BN_GRU, H), jnp.float32),
            pltpu.VMEM((BN_GRU, H), jnp.float32),
        ],
        compiler_params=pltpu.CompilerParams(
            dimension_semantics=("parallel", "arbitrary")),
    )(xt, Wih0T, Whh0T, bih0, bhh0, Wih1T, Whh1T, bih1, bhh1, Wg0T)


def _sc_conv_body(m_hbm, src_hbm, dst_hbm, p_hbm, src_v, dst_v, rows_v, zb,
                  acc, *sems):
    sems_g = sems[:NBUF]
    sems_s = sems[NBUF:]
    c = lax.axis_index("c")
    s = lax.axis_index("s")
    w = c * NS + s
    nch = jnp.where(c == 0, F0, F1)

    zeros16 = jnp.zeros((16,), jnp.float32)
    for r in range(16):
        for cc in range(H // 16):
            zb[r, pl.ds(cc * 16, 16)] = zeros16

    def zero_acc(i, carry):
        pltpu.sync_copy(zb, acc.at[pl.ds(s * ROWS_PER_TILE_Z + i * 16, 16)])
        return carry
    lax.fori_loop(0, ROWS_PER_TILE_Z // 16, zero_acc, 0)
    plsc.subcore_barrier()

    def stage(j, b):
        pltpu.make_async_copy(m_hbm.at[src_v.at[j]], rows_v.at[b],
                              sems_g[b]).wait()
        pltpu.async_copy(rows_v.at[b], acc.at[dst_v.at[j]], sems_s[b],
                         add=True)
        pltpu.make_async_copy(rows_v.at[b], acc.at[dst_v.at[j]],
                              sems_s[b]).wait()

    for hh in range(NPH):
        @pl.when(hh * PH < nch)
        def _():
            pltpu.sync_copy(src_hbm.at[w, pl.ds(hh * PH, PH)], src_v)
            pltpu.sync_copy(dst_hbm.at[w, pl.ds(hh * PH, PH)], dst_v)
            for b in range(NBUF):
                pltpu.async_copy(m_hbm.at[src_v.at[b]], rows_v.at[b],
                                 sems_g[b])

            def body(jj, carry):
                for b in range(NBUF):
                    j = jj * NBUF + b
                    stage(j, b)
                    pltpu.async_copy(m_hbm.at[src_v.at[j + NBUF]],
                                     rows_v.at[b], sems_g[b])
                return carry
            lax.fori_loop(0, PH // NBUF - 1, body, 0)
            for b in range(NBUF):
                stage(PH - NBUF + b, b)
    plsc.subcore_barrier()

    base = s * ROWS_PER_TILE_Z
    pltpu.sync_copy(acc.at[pl.ds(base, ROWS_PER_TILE_Z)],
                    p_hbm.at[c, pl.ds(base, ROWS_PER_TILE_Z)])


def _sc_deg_body(dst_hbm, deg_hbm, dst_v, ones_v, zline, dacc):
    w = lax.axis_index("s")

    ones16 = jnp.ones((16,), jnp.float32)
    zeros16 = jnp.zeros((16,), jnp.float32)
    for cc in range(CH // 16):
        ones_v[pl.ds(cc * 16, 16)] = ones16
        zline[pl.ds(cc * 16, 16)] = zeros16

    def zero_deg(i, carry):
        pltpu.sync_copy(zline, dacc.at[pl.ds(w * ROWS_PER_TILE_Z + i * CH, CH)])
        return carry
    lax.fori_loop(0, ROWS_PER_TILE_Z // CH, zero_deg, 0)
    plsc.subcore_barrier()

    pltpu.sync_copy(dst_hbm.at[w], dst_v)

    def chunk(j, carry):
        pltpu.sync_copy(ones_v, dacc.at[dst_v.at[j]], add=True)
        return carry
    lax.fori_loop(0, NCHUNK, chunk, 0)
    plsc.subcore_barrier()

    @pl.when(w == 0)
    def _():
        pltpu.sync_copy(dacc, deg_hbm)


def _sc_mesh(num_cores):
    return plsc.VectorSubcoreMesh(core_axis_name="c", subcore_axis_name="s",
                                  num_cores=num_cores, num_subcores=NS)


@functools.lru_cache(maxsize=None)
def _make_sc_conv(tag):
    return pl.kernel(
        _sc_conv_body,
        name=f"sc_conv_{tag}",
        out_type=jax.ShapeDtypeStruct((NC, NPAD, H), jnp.float32),
        mesh=_sc_mesh(NC),
        scratch_types=[
            pltpu.VMEM((PH, CH), jnp.int32),
            pltpu.VMEM((PH, CH), jnp.int32),
            pltpu.VMEM((NBUF, CH, H), jnp.float32),
            pltpu.VMEM((16, H), jnp.float32),
            pltpu.VMEM_SHARED((NPAD, H), jnp.float32),
        ] + [pltpu.SemaphoreType.DMA] * (2 * NBUF),
    )


@functools.lru_cache(maxsize=None)
def _make_sc_deg():
    return pl.kernel(
        _sc_deg_body,
        out_type=jax.ShapeDtypeStruct((NPAD,), jnp.float32),
        mesh=_sc_mesh(1),
        scratch_types=[
            pltpu.VMEM((NCHUNK, CH), jnp.int32),
            pltpu.VMEM((CH,), jnp.float32),
            pltpu.VMEM((CH,), jnp.float32),
            pltpu.VMEM_SHARED((NPAD,), jnp.float32),
        ],
    )


def _mid_body(p_ref, degt_ref, bg_ref, wg_ref, m_ref):
    dsum = degt_ref[...]
    inv = jnp.where(dsum > 0.0, 1.0 / jnp.where(dsum > 0.0, dsum, 1.0), 0.0)
    out = jax.nn.relu((p_ref[0] + p_ref[1]) * inv + bg_ref[...])
    m_ref[...] = jnp.dot(out, wg_ref[...], preferred_element_type=jnp.float32)


def _run_mid_tc(p, degt, bg0, Wg1T):
    nb = N // BN_D
    full = lambda i: (0, 0)
    return pl.pallas_call(
        _mid_body,
        grid=(nb,),
        in_specs=[
            pl.BlockSpec((NC, BN_D, H), lambda i: (0, i, 0)),
            pl.BlockSpec((BN_D, 1), lambda i: (i, 0)),
            pl.BlockSpec((1, H), full),
            pl.BlockSpec((H, H), full),
        ],
        out_specs=pl.BlockSpec((BN_D, H), lambda i: (i, 0)),
        out_shape=jax.ShapeDtypeStruct((N, H), jnp.float32),
        compiler_params=pltpu.CompilerParams(
            dimension_semantics=("parallel",)),
    )(p, degt, bg0, Wg1T)


def _final_body(q_ref, degt_ref, h_ref, bg_ref, wskip_ref, bskip_ref,
                gamma_ref, beta_ref, wout_ref, bout_ref, y_ref):
    dsum = degt_ref[...]
    inv = jnp.where(dsum > 0.0, 1.0 / jnp.where(dsum > 0.0, dsum, 1.0), 0.0)
    out = jax.nn.relu((q_ref[0] + q_ref[1]) * inv + bg_ref[...])
    res = out + jnp.dot(h_ref[...], wskip_ref[...],
                        preferred_element_type=jnp.float32) + bskip_ref[...]
    mu = jnp.mean(res, axis=-1, keepdims=True)
    var = jnp.mean((res - mu) * (res - mu), axis=-1, keepdims=True)
    ln = gamma_ref[...] * (res - mu) / jnp.sqrt(var + 1e-5) + beta_ref[...]
    y_ref[...] = jnp.dot(ln, wout_ref[...],
                         preferred_element_type=jnp.float32) + bout_ref[...]


def _run_final_tc(q, degt, h, bg1, WskipT, bskip, gamma, beta, WoutT, bout):
    nb = N // BN_D
    full = lambda i: (0, 0)
    return pl.pallas_call(
        _final_body,
        grid=(nb,),
        in_specs=[
            pl.BlockSpec((NC, BN_D, H), lambda i: (0, i, 0)),
            pl.BlockSpec((BN_D, 1), lambda i: (i, 0)),
            pl.BlockSpec((BN_D, H), lambda i: (i, 0)),
            pl.BlockSpec((1, H), full),
            pl.BlockSpec((H, H), full),
            pl.BlockSpec((1, H), full),
            pl.BlockSpec((1, H), full),
            pl.BlockSpec((1, H), full),
            pl.BlockSpec((H, OUT), full),
            pl.BlockSpec((1, OUT), full),
        ],
        out_specs=pl.BlockSpec((BN_D, OUT), lambda i: (i, 0)),
        out_shape=jax.ShapeDtypeStruct((N, OUT), jnp.float32),
        compiler_params=pltpu.CompilerParams(
            dimension_semantics=("parallel",)),
    )(q, degt, h, bg1, WskipT, bskip, gamma, beta, WoutT, bout)


def kernel(x, edge_index, Wih0, Whh0, bih0, bhh0, Wih1, Whh1, bih1, bhh1,
           Wg0, bg0, Wg1, bg1, Wskip, bskip, gamma, beta, Wout, bout):
    src = edge_index[0]
    dst = edge_index[1]
    pad = EPAD - E

    dstp_deg = jnp.concatenate([dst, jnp.full((pad,), N, jnp.int32)]
                               ).reshape(NS, NCHUNK, CH)

    def asym(v, fill):
        a0 = v[:E0].reshape(NS, F0, CH)
        a1 = jnp.concatenate([v[E0:], jnp.full((pad,), fill, jnp.int32)]
                             ).reshape(NS, F1, CH)
        if FMAX > F1:
            a1 = jnp.concatenate(
                [a1, jnp.full((NS, FMAX - F1, CH), fill, jnp.int32)], axis=1)
        if FMAX > F0:
            a0 = jnp.concatenate(
                [a0, jnp.full((NS, FMAX - F0, CH), fill, jnp.int32)], axis=1)
        return jnp.concatenate([a0, a1], axis=0)
    srcp = asym(src, 0)
    dstp = asym(dst, N)

    xt = jnp.transpose(x, (1, 0, 2))

    deg_flat = _make_sc_deg()(dstp_deg)
    degt = deg_flat[:N].reshape(N, 1)

    row = lambda v: v.reshape(1, -1)
    h, m0 = _run_gru_tc(xt, Wih0.T, Whh0.T, row(bih0), row(bhh0),
                        Wih1.T, Whh1.T, row(bih1), row(bhh1), Wg0.T)

    p = _make_sc_conv(0)(m0, srcp, dstp)
    m1 = _run_mid_tc(p, degt, row(bg0), Wg1.T)
    q = _make_sc_conv(1)(m1, srcp, dstp)

    y = _run_final_tc(q, degt, h, row(bg1), Wskip.T, row(bskip),
                      row(gamma), row(beta), Wout.T, row(bout))
    return y


# speedup vs baseline: 2.5977x; 2.5977x over previous
import functools

import jax
import jax.numpy as jnp
from jax import lax
from jax.experimental import pallas as pl
from jax.experimental.pallas import tpu as pltpu
from jax.experimental.pallas import tpu_sc as plsc

N = 10000
T = 12
H = 128
OUT = 32
E = 320000

NC = 2
NS = 16
NW = NC * NS
CH = 128
NCHUNK = 80
EPAD = NW * NCHUNK * CH
NPAD = 10240
ROWS_PER_TILE_Z = NPAD // NS

BN_GRU = 400
BN_D = 1000


def _gru_body(x_ref, wih0, whh0, bih0, bhh0, wih1, whh1, bih1, bhh1, wg0,
              h_ref, m_ref, h1_scr, h2_scr):
    t = pl.program_id(1)

    @pl.when(t == 0)
    def _():
        h1_scr[...] = jnp.zeros_like(h1_scr)
        h2_scr[...] = jnp.zeros_like(h2_scr)

    xt = x_ref[0]

    def step(xin, h, wih, whh, bih, bhh):
        gi = jnp.dot(xin, wih[...], preferred_element_type=jnp.float32) + bih[...]
        gh = jnp.dot(h, whh[...], preferred_element_type=jnp.float32) + bhh[...]
        r = jax.nn.sigmoid(gi[:, :H] + gh[:, :H])
        z = jax.nn.sigmoid(gi[:, H:2 * H] + gh[:, H:2 * H])
        n = jnp.tanh(gi[:, 2 * H:] + r * gh[:, 2 * H:])
        return (1.0 - z) * n + z * h

    h1 = step(xt, h1_scr[...], wih0, whh0, bih0, bhh0)
    h2 = step(h1, h2_scr[...], wih1, whh1, bih1, bhh1)
    h1_scr[...] = h1
    h2_scr[...] = h2

    @pl.when(t == T - 1)
    def _():
        h_ref[...] = h2
        m_ref[...] = jnp.dot(h2, wg0[...], preferred_element_type=jnp.float32)


def _run_gru_tc(xt, Wih0T, Whh0T, bih0, bhh0, Wih1T, Whh1T, bih1, bhh1, Wg0T):
    nb = pl.cdiv(N, BN_GRU)
    full = lambda i, t: (0, 0)
    return pl.pallas_call(
        _gru_body,
        grid=(nb, T),
        in_specs=[
            pl.BlockSpec((1, BN_GRU, H), lambda i, t: (t, i, 0)),
            pl.BlockSpec((H, 3 * H), full),
            pl.BlockSpec((H, 3 * H), full),
            pl.BlockSpec((1, 3 * H), full),
            pl.BlockSpec((1, 3 * H), full),
            pl.BlockSpec((H, 3 * H), full),
            pl.BlockSpec((H, 3 * H), full),
            pl.BlockSpec((1, 3 * H), full),
            pl.BlockSpec((1, 3 * H), full),
            pl.BlockSpec((H, H), full),
        ],
        out_specs=[
            pl.BlockSpec((BN_GRU, H), lambda i, t: (i, 0)),
            pl.BlockSpec((BN_GRU, H), lambda i, t: (i, 0)),
        ],
        out_shape=[
            jax.ShapeDtypeStruct((N, H), jnp.float32),
            jax.ShapeDtypeStruct((N, H), jnp.float32),
        ],
        scratch_shapes=[
            pltpu.VMEM((BN_GRU, H), jnp.float32),
            pltpu.VMEM((BN_GRU, H), jnp.float32),
        ],
        compiler_params=pltpu.CompilerParams(
            dimension_semantics=("parallel", "arbitrary")),
    )(xt, Wih0T, Whh0T, bih0, bhh0, Wih1T, Whh1T, bih1, bhh1, Wg0T)


def _sc_conv_body(m_hbm, src_hbm, dst_hbm, p_hbm, src_v, dst_v, rows_v, zb,
                  acc, sem):
    c = lax.axis_index("c")
    s = lax.axis_index("s")
    w = c * NS + s

    zeros16 = jnp.zeros((16,), jnp.float32)
    for r in range(16):
        for cc in range(H // 16):
            zb[r, pl.ds(cc * 16, 16)] = zeros16

    def zero_acc(i, carry):
        pltpu.sync_copy(zb, acc.at[pl.ds(s * ROWS_PER_TILE_Z + i * 16, 16)])
        return carry
    lax.fori_loop(0, ROWS_PER_TILE_Z // 16, zero_acc, 0)
    plsc.subcore_barrier()

    pltpu.sync_copy(src_hbm.at[w], src_v)
    pltpu.sync_copy(dst_hbm.at[w], dst_v)

    def chunk(j, carry):
        pltpu.async_copy(m_hbm.at[src_v.at[j]], rows_v, sem).wait()
        pltpu.sync_copy(rows_v, acc.at[dst_v.at[j]], add=True)
        return carry
    lax.fori_loop(0, NCHUNK, chunk, 0)
    plsc.subcore_barrier()

    base = s * ROWS_PER_TILE_Z
    pltpu.sync_copy(acc.at[pl.ds(base, ROWS_PER_TILE_Z)],
                    p_hbm.at[c, pl.ds(base, ROWS_PER_TILE_Z)])


def _sc_deg_body(dst_hbm, deg_hbm, dst_v, ones_v, zline, dacc):
    w = lax.axis_index("s")

    ones16 = jnp.ones((16,), jnp.float32)
    zeros16 = jnp.zeros((16,), jnp.float32)
    for cc in range(CH // 16):
        ones_v[pl.ds(cc * 16, 16)] = ones16
        zline[pl.ds(cc * 16, 16)] = zeros16

    def zero_deg(i, carry):
        pltpu.sync_copy(zline, dacc.at[pl.ds(w * ROWS_PER_TILE_Z + i * CH, CH)])
        return carry
    lax.fori_loop(0, ROWS_PER_TILE_Z // CH, zero_deg, 0)
    plsc.subcore_barrier()

    pltpu.sync_copy(dst_hbm.at[w], dst_v)

    def chunk(j, carry):
        pltpu.sync_copy(ones_v, dacc.at[dst_v.at[j]], add=True)
        return carry
    lax.fori_loop(0, NC * NCHUNK, chunk, 0)
    plsc.subcore_barrier()

    @pl.when(w == 0)
    def _():
        pltpu.sync_copy(dacc, deg_hbm)


def _sc_mesh(num_cores):
    return plsc.VectorSubcoreMesh(core_axis_name="c", subcore_axis_name="s",
                                  num_cores=num_cores, num_subcores=NS)


@functools.lru_cache(maxsize=None)
def _make_sc_conv(tag):
    return pl.kernel(
        _sc_conv_body,
        name=f"sc_conv_{tag}",
        out_type=jax.ShapeDtypeStruct((NC, NPAD, H), jnp.float32),
        mesh=_sc_mesh(NC),
        scratch_types=[
            pltpu.VMEM((NCHUNK, CH), jnp.int32),
            pltpu.VMEM((NCHUNK, CH), jnp.int32),
            pltpu.VMEM((CH, H), jnp.float32),
            pltpu.VMEM((16, H), jnp.float32),
            pltpu.VMEM_SHARED((NPAD, H), jnp.float32),
            pltpu.SemaphoreType.DMA,
        ],
    )


@functools.lru_cache(maxsize=None)
def _make_sc_deg():
    return pl.kernel(
        _sc_deg_body,
        out_type=jax.ShapeDtypeStruct((NPAD,), jnp.float32),
        mesh=_sc_mesh(1),
        scratch_types=[
            pltpu.VMEM((NC * NCHUNK, CH), jnp.int32),
            pltpu.VMEM((CH,), jnp.float32),
            pltpu.VMEM((CH,), jnp.float32),
            pltpu.VMEM_SHARED((NPAD,), jnp.float32),
        ],
    )


def _mid_body(p_ref, degt_ref, bg_ref, wg_ref, m_ref):
    dsum = degt_ref[...]
    inv = jnp.where(dsum > 0.0, 1.0 / jnp.where(dsum > 0.0, dsum, 1.0), 0.0)
    out = jax.nn.relu((p_ref[0] + p_ref[1]) * inv + bg_ref[...])
    m_ref[...] = jnp.dot(out, wg_ref[...], preferred_element_type=jnp.float32)


def _run_mid_tc(p, degt, bg0, Wg1T):
    nb = N // BN_D
    full = lambda i: (0, 0)
    return pl.pallas_call(
        _mid_body,
        grid=(nb,),
        in_specs=[
            pl.BlockSpec((NC, BN_D, H), lambda i: (0, i, 0)),
            pl.BlockSpec((BN_D, 1), lambda i: (i, 0)),
            pl.BlockSpec((1, H), full),
            pl.BlockSpec((H, H), full),
        ],
        out_specs=pl.BlockSpec((BN_D, H), lambda i: (i, 0)),
        out_shape=jax.ShapeDtypeStruct((N, H), jnp.float32),
        compiler_params=pltpu.CompilerParams(
            dimension_semantics=("parallel",)),
    )(p, degt, bg0, Wg1T)


def _final_body(q_ref, degt_ref, h_ref, bg_ref, wskip_ref, bskip_ref,
                gamma_ref, beta_ref, wout_ref, bout_ref, y_ref):
    dsum = degt_ref[...]
    inv = jnp.where(dsum > 0.0, 1.0 / jnp.where(dsum > 0.0, dsum, 1.0), 0.0)
    out = jax.nn.relu((q_ref[0] + q_ref[1]) * inv + bg_ref[...])
    res = out + jnp.dot(h_ref[...], wskip_ref[...],
                        preferred_element_type=jnp.float32) + bskip_ref[...]
    mu = jnp.mean(res, axis=-1, keepdims=True)
    var = jnp.mean((res - mu) * (res - mu), axis=-1, keepdims=True)
    ln = gamma_ref[...] * (res - mu) / jnp.sqrt(var + 1e-5) + beta_ref[...]
    y_ref[...] = jnp.dot(ln, wout_ref[...],
                         preferred_element_type=jnp.float32) + bout_ref[...]


def _run_final_tc(q, degt, h, bg1, WskipT, bskip, gamma, beta, WoutT, bout):
    nb = N // BN_D
    full = lambda i: (0, 0)
    return pl.pallas_call(
        _final_body,
        grid=(nb,),
        in_specs=[
            pl.BlockSpec((NC, BN_D, H), lambda i: (0, i, 0)),
            pl.BlockSpec((BN_D, 1), lambda i: (i, 0)),
            pl.BlockSpec((BN_D, H), lambda i: (i, 0)),
            pl.BlockSpec((1, H), full),
            pl.BlockSpec((H, H), full),
            pl.BlockSpec((1, H), full),
            pl.BlockSpec((1, H), full),
            pl.BlockSpec((1, H), full),
            pl.BlockSpec((H, OUT), full),
            pl.BlockSpec((1, OUT), full),
        ],
        out_specs=pl.BlockSpec((BN_D, OUT), lambda i: (i, 0)),
        out_shape=jax.ShapeDtypeStruct((N, OUT), jnp.float32),
        compiler_params=pltpu.CompilerParams(
            dimension_semantics=("parallel",)),
    )(q, degt, h, bg1, WskipT, bskip, gamma, beta, WoutT, bout)


def kernel(x, edge_index, Wih0, Whh0, bih0, bhh0, Wih1, Whh1, bih1, bhh1,
           Wg0, bg0, Wg1, bg1, Wskip, bskip, gamma, beta, Wout, bout):
    src = edge_index[0]
    dst = edge_index[1]
    pad = EPAD - E

    srcp = jnp.concatenate([src, jnp.zeros((pad,), jnp.int32)]
                           ).reshape(NW, NCHUNK, CH)
    dstp = jnp.concatenate([dst, jnp.full((pad,), N, jnp.int32)]
                           ).reshape(NW, NCHUNK, CH)
    dstp_deg = dstp.reshape(NS, NC * NCHUNK, CH)

    xt = jnp.transpose(x, (1, 0, 2))

    deg_flat = _make_sc_deg()(dstp_deg)
    degt = deg_flat[:N].reshape(N, 1)

    row = lambda v: v.reshape(1, -1)
    h, m0 = _run_gru_tc(xt, Wih0.T, Whh0.T, row(bih0), row(bhh0),
                        Wih1.T, Whh1.T, row(bih1), row(bhh1), Wg0.T)

    p = _make_sc_conv(0)(m0, srcp, dstp)
    m1 = _run_mid_tc(p, degt, row(bg0), Wg1.T)
    q = _make_sc_conv(1)(m1, srcp, dstp)

    y = _run_final_tc(q, degt, h, row(bg1), Wskip.T, row(bskip),
                      row(gamma), row(beta), Wout.T, row(bout))
    return y


# speedup vs baseline: 3.0105x; 1.1589x over previous
import functools

import jax
import jax.numpy as jnp
from jax import lax
from jax.experimental import pallas as pl
from jax.experimental.pallas import tpu as pltpu
from jax.experimental.pallas import tpu_sc as plsc

N = 10000
T = 12
H = 128
OUT = 32
E = 320000

NC = 2
NS = 16
NW = NC * NS
CH = 128
NCHUNK = 79
EPAD = NW * NCHUNK * CH
NPAD = 10240
ROWS_PER_TILE_Z = NPAD // NS

BN_GRU = 400
BN_D = 1000


def _gru_body(x_ref, wih0, whh0, bih0, bhh0, wih1, whh1, bih1, bhh1, wg0,
              h_ref, m_ref, h1_scr, h2_scr):
    t = pl.program_id(1)

    @pl.when(t == 0)
    def _():
        h1_scr[...] = jnp.zeros_like(h1_scr)
        h2_scr[...] = jnp.zeros_like(h2_scr)

    xt = x_ref[0]

    def step(xin, h, wih, whh, bih, bhh):
        gi = jnp.dot(xin, wih[...], preferred_element_type=jnp.float32) + bih[...]
        gh = jnp.dot(h, whh[...], preferred_element_type=jnp.float32) + bhh[...]
        r = jax.nn.sigmoid(gi[:, :H] + gh[:, :H])
        z = jax.nn.sigmoid(gi[:, H:2 * H] + gh[:, H:2 * H])
        n = jnp.tanh(gi[:, 2 * H:] + r * gh[:, 2 * H:])
        return (1.0 - z) * n + z * h

    h1 = step(xt, h1_scr[...], wih0, whh0, bih0, bhh0)
    h2 = step(h1, h2_scr[...], wih1, whh1, bih1, bhh1)
    h1_scr[...] = h1
    h2_scr[...] = h2

    @pl.when(t == T - 1)
    def _():
        h_ref[...] = h2
        m_ref[...] = jnp.dot(h2, wg0[...], preferred_element_type=jnp.float32)


def _run_gru_tc(xt, Wih0T, Whh0T, bih0, bhh0, Wih1T, Whh1T, bih1, bhh1, Wg0T):
    nb = pl.cdiv(N, BN_GRU)
    full = lambda i, t: (0, 0)
    return pl.pallas_call(
        _gru_body,
        grid=(nb, T),
        in_specs=[
            pl.BlockSpec((1, BN_GRU, H), lambda i, t: (t, i, 0)),
            pl.BlockSpec((H, 3 * H), full),
            pl.BlockSpec((H, 3 * H), full),
            pl.BlockSpec((1, 3 * H), full),
            pl.BlockSpec((1, 3 * H), full),
            pl.BlockSpec((H, 3 * H), full),
            pl.BlockSpec((H, 3 * H), full),
            pl.BlockSpec((1, 3 * H), full),
            pl.BlockSpec((1, 3 * H), full),
            pl.BlockSpec((H, H), full),
        ],
        out_specs=[
            pl.BlockSpec((BN_GRU, H), lambda i, t: (i, 0)),
            pl.BlockSpec((BN_GRU, H), lambda i, t: (i, 0)),
        ],
        out_shape=[
            jax.ShapeDtypeStruct((N, H), jnp.float32),
            jax.ShapeDtypeStruct((N, H), jnp.float32),
        ],
        scratch_shapes=[
            pltpu.VMEM((BN_GRU, H), jnp.float32),
            pltpu.VMEM((BN_GRU, H), jnp.float32),
        ],
        compiler_params=pltpu.CompilerParams(
            dimension_semantics=("parallel", "arbitrary")),
    )(xt, Wih0T, Whh0T, bih0, bhh0, Wih1T, Whh1T, bih1, bhh1, Wg0T)


def _sc_conv_body(compute_deg, m_hbm, src_hbm, dst_hbm, *rest):
    if compute_deg:
        (p_hbm, deg_hbm, src_v, dst_v, rows_v, zb, ones_v, zline, acc, dacc,
         sem) = rest
    else:
        (p_hbm, src_v, dst_v, rows_v, zb, ones_v, zline, acc, dacc,
         sem) = rest
    c = lax.axis_index("c")
    s = lax.axis_index("s")
    w = c * NS + s

    zeros16 = jnp.zeros((16,), jnp.float32)
    ones16 = jnp.ones((16,), jnp.float32)
    for r in range(16):
        for cc in range(H // 16):
            zb[r, pl.ds(cc * 16, 16)] = zeros16
    for cc in range(CH // 16):
        ones_v[pl.ds(cc * 16, 16)] = ones16
        zline[pl.ds(cc * 16, 16)] = zeros16

    def zero_acc(i, carry):
        pltpu.sync_copy(zb, acc.at[pl.ds(s * ROWS_PER_TILE_Z + i * 16, 16)])
        return carry
    lax.fori_loop(0, ROWS_PER_TILE_Z // 16, zero_acc, 0)
    if compute_deg:
        def zero_deg(i, carry):
            pltpu.sync_copy(zline,
                            dacc.at[pl.ds(s * ROWS_PER_TILE_Z + i * CH, CH)])
            return carry
        lax.fori_loop(0, ROWS_PER_TILE_Z // CH, zero_deg, 0)
    plsc.subcore_barrier()

    pltpu.sync_copy(src_hbm.at[w], src_v)
    pltpu.sync_copy(dst_hbm.at[w], dst_v)

    def chunk(j, carry):
        idx_d = dst_v.at[j]
        pltpu.async_copy(m_hbm.at[src_v.at[j]], rows_v, sem).wait()
        pltpu.sync_copy(rows_v, acc.at[idx_d], add=True)
        if compute_deg:
            pltpu.sync_copy(ones_v, dacc.at[idx_d], add=True)
        return carry
    lax.fori_loop(0, NCHUNK, chunk, 0)
    plsc.subcore_barrier()

    base = s * ROWS_PER_TILE_Z
    pltpu.sync_copy(acc.at[pl.ds(base, ROWS_PER_TILE_Z)],
                    p_hbm.at[c, pl.ds(base, ROWS_PER_TILE_Z)])
    if compute_deg:
        @pl.when(s == 0)
        def _():
            pltpu.sync_copy(dacc, deg_hbm.at[pl.ds(c * NPAD, NPAD)])


def _sc_mesh(num_cores):
    return plsc.VectorSubcoreMesh(core_axis_name="c", subcore_axis_name="s",
                                  num_cores=num_cores, num_subcores=NS)


@functools.lru_cache(maxsize=None)
def _make_sc_conv(compute_deg):
    if compute_deg:
        out_type = (jax.ShapeDtypeStruct((NC, NPAD, H), jnp.float32),
                    jax.ShapeDtypeStruct((NC * NPAD,), jnp.float32))
    else:
        out_type = jax.ShapeDtypeStruct((NC, NPAD, H), jnp.float32)
    return pl.kernel(
        functools.partial(_sc_conv_body, compute_deg),
        name=f"sc_conv_{int(compute_deg)}",
        out_type=out_type,
        mesh=_sc_mesh(NC),
        scratch_types=[
            pltpu.VMEM((NCHUNK, CH), jnp.int32),
            pltpu.VMEM((NCHUNK, CH), jnp.int32),
            pltpu.VMEM((CH, H), jnp.float32),
            pltpu.VMEM((16, H), jnp.float32),
            pltpu.VMEM((CH,), jnp.float32),
            pltpu.VMEM((CH,), jnp.float32),
            pltpu.VMEM_SHARED((NPAD, H), jnp.float32),
            pltpu.VMEM_SHARED((NPAD,), jnp.float32),
            pltpu.SemaphoreType.DMA,
        ],
    )


def _mid_body(p_ref, degt_ref, bg_ref, wg_ref, m_ref):
    dsum = degt_ref[...]
    inv = jnp.where(dsum > 0.0, 1.0 / jnp.where(dsum > 0.0, dsum, 1.0), 0.0)
    out = jax.nn.relu((p_ref[0] + p_ref[1]) * inv + bg_ref[...])
    m_ref[...] = jnp.dot(out, wg_ref[...], preferred_element_type=jnp.float32)


def _run_mid_tc(p, degt, bg0, Wg1T):
    nb = N // BN_D
    full = lambda i: (0, 0)
    return pl.pallas_call(
        _mid_body,
        grid=(nb,),
        in_specs=[
            pl.BlockSpec((NC, BN_D, H), lambda i: (0, i, 0)),
            pl.BlockSpec((BN_D, 1), lambda i: (i, 0)),
            pl.BlockSpec((1, H), full),
            pl.BlockSpec((H, H), full),
        ],
        out_specs=pl.BlockSpec((BN_D, H), lambda i: (i, 0)),
        out_shape=jax.ShapeDtypeStruct((N, H), jnp.float32),
        compiler_params=pltpu.CompilerParams(
            dimension_semantics=("parallel",)),
    )(p, degt, bg0, Wg1T)


def _final_body(q_ref, degt_ref, h_ref, bg_ref, wskip_ref, bskip_ref,
                gamma_ref, beta_ref, wout_ref, bout_ref, y_ref):
    dsum = degt_ref[...]
    inv = jnp.where(dsum > 0.0, 1.0 / jnp.where(dsum > 0.0, dsum, 1.0), 0.0)
    out = jax.nn.relu((q_ref[0] + q_ref[1]) * inv + bg_ref[...])
    res = out + jnp.dot(h_ref[...], wskip_ref[...],
                        preferred_element_type=jnp.float32) + bskip_ref[...]
    mu = jnp.mean(res, axis=-1, keepdims=True)
    var = jnp.mean((res - mu) * (res - mu), axis=-1, keepdims=True)
    ln = gamma_ref[...] * (res - mu) / jnp.sqrt(var + 1e-5) + beta_ref[...]
    y_ref[...] = jnp.dot(ln, wout_ref[...],
                         preferred_element_type=jnp.float32) + bout_ref[...]


def _run_final_tc(q, degt, h, bg1, WskipT, bskip, gamma, beta, WoutT, bout):
    nb = N // BN_D
    full = lambda i: (0, 0)
    return pl.pallas_call(
        _final_body,
        grid=(nb,),
        in_specs=[
            pl.BlockSpec((NC, BN_D, H), lambda i: (0, i, 0)),
            pl.BlockSpec((BN_D, 1), lambda i: (i, 0)),
            pl.BlockSpec((BN_D, H), lambda i: (i, 0)),
            pl.BlockSpec((1, H), full),
            pl.BlockSpec((H, H), full),
            pl.BlockSpec((1, H), full),
            pl.BlockSpec((1, H), full),
            pl.BlockSpec((1, H), full),
            pl.BlockSpec((H, OUT), full),
            pl.BlockSpec((1, OUT), full),
        ],
        out_specs=pl.BlockSpec((BN_D, OUT), lambda i: (i, 0)),
        out_shape=jax.ShapeDtypeStruct((N, OUT), jnp.float32),
        compiler_params=pltpu.CompilerParams(
            dimension_semantics=("parallel",)),
    )(q, degt, h, bg1, WskipT, bskip, gamma, beta, WoutT, bout)


def kernel(x, edge_index, Wih0, Whh0, bih0, bhh0, Wih1, Whh1, bih1, bhh1,
           Wg0, bg0, Wg1, bg1, Wskip, bskip, gamma, beta, Wout, bout):
    src = edge_index[0]
    dst = edge_index[1]
    pad = EPAD - E

    srcp = jnp.concatenate([src, jnp.zeros((pad,), jnp.int32)]
                           ).reshape(NW, NCHUNK, CH)
    dstp = jnp.concatenate([dst, jnp.full((pad,), N, jnp.int32)]
                           ).reshape(NW, NCHUNK, CH)

    xt = jnp.transpose(x, (1, 0, 2))

    row = lambda v: v.reshape(1, -1)
    h, m0 = _run_gru_tc(xt, Wih0.T, Whh0.T, row(bih0), row(bhh0),
                        Wih1.T, Whh1.T, row(bih1), row(bhh1), Wg0.T)

    p, deg_flat = _make_sc_conv(True)(m0, srcp, dstp)
    degt = (deg_flat[:N] + deg_flat[NPAD:NPAD + N]).reshape(N, 1)

    m1 = _run_mid_tc(p, degt, row(bg0), Wg1.T)
    q = _make_sc_conv(False)(m1, srcp, dstp)

    y = _run_final_tc(q, degt, h, row(bg1), Wskip.T, row(bskip),
                      row(gamma), row(beta), Wout.T, row(bout))
    return y


# speedup vs baseline: 3.3733x; 1.1205x over previous
import functools

import jax
import jax.numpy as jnp
from jax import lax
from jax.experimental import pallas as pl
from jax.experimental.pallas import tpu as pltpu
from jax.experimental.pallas import tpu_sc as plsc

N = 10000
T = 12
H = 128
OUT = 32
E = 320000

NC = 2
NS = 16
NW = NC * NS
CH = 128
NCHUNK = 79
EPAD = NW * NCHUNK * CH
NPAD = 10240
ROWS_PER_TILE_Z = NPAD // NS

BN_GRU = 400
BN_D = 1000


def _gru_body(x_ref, wrz0, wni0, wnh0, brz0, bni0, bnh0,
              wrz1, wni1, wnh1, brz1, bni1, bnh1, wg0,
              h_ref, m_ref):
    def step(xin, h, wrz, wni, wnh, brz, bni, bnh):
        cat = jnp.concatenate([xin, h], axis=1)
        rz = jax.nn.sigmoid(
            jnp.dot(cat, wrz[...], preferred_element_type=jnp.float32)
            + brz[...])
        r = rz[:, :H]
        z = rz[:, H:]
        i_n = jnp.dot(xin, wni[...], preferred_element_type=jnp.float32) + bni[...]
        h_n = jnp.dot(h, wnh[...], preferred_element_type=jnp.float32) + bnh[...]
        n = jnp.tanh(i_n + r * h_n)
        return (1.0 - z) * n + z * h

    h1 = jnp.zeros((x_ref.shape[0], H), jnp.float32)
    h2 = jnp.zeros((x_ref.shape[0], H), jnp.float32)
    for t in range(T):
        xt = x_ref[:, t * H:(t + 1) * H]
        h1 = step(xt, h1, wrz0, wni0, wnh0, brz0, bni0, bnh0)
        h2 = step(h1, h2, wrz1, wni1, wnh1, brz1, bni1, bnh1)
    h_ref[...] = h2
    m_ref[...] = jnp.dot(h2, wg0[...], preferred_element_type=jnp.float32)


def _run_gru_tc(x2, Wih0T, Whh0T, bih0, bhh0, Wih1T, Whh1T, bih1, bhh1, Wg0T):
    nb = N // BN_GRU
    full = lambda i: (0, 0)

    def prep(wihT, whhT, bih, bhh):
        wrz = jnp.concatenate([wihT[:, :2 * H], whhT[:, :2 * H]], axis=0)
        return (wrz, wihT[:, 2 * H:], whhT[:, 2 * H:],
                (bih[:, :2 * H] + bhh[:, :2 * H]), bih[:, 2 * H:],
                bhh[:, 2 * H:])
    wrz0, wni0, wnh0, brz0, bni0, bnh0 = prep(Wih0T, Whh0T, bih0, bhh0)
    wrz1, wni1, wnh1, brz1, bni1, bnh1 = prep(Wih1T, Whh1T, bih1, bhh1)

    wspec = [
        pl.BlockSpec((2 * H, 2 * H), full),
        pl.BlockSpec((H, H), full),
        pl.BlockSpec((H, H), full),
        pl.BlockSpec((1, 2 * H), full),
        pl.BlockSpec((1, H), full),
        pl.BlockSpec((1, H), full),
    ]
    return pl.pallas_call(
        _gru_body,
        grid=(nb,),
        in_specs=[pl.BlockSpec((BN_GRU, T * H), lambda i: (i, 0))]
        + wspec + wspec + [pl.BlockSpec((H, H), full)],
        out_specs=[
            pl.BlockSpec((BN_GRU, H), lambda i: (i, 0)),
            pl.BlockSpec((BN_GRU, H), lambda i: (i, 0)),
        ],
        out_shape=[
            jax.ShapeDtypeStruct((N, H), jnp.float32),
            jax.ShapeDtypeStruct((N, H), jnp.float32),
        ],
        compiler_params=pltpu.CompilerParams(
            dimension_semantics=("parallel",)),
    )(x2, wrz0, wni0, wnh0, brz0, bni0, bnh0,
      wrz1, wni1, wnh1, brz1, bni1, bnh1, Wg0T)


def _sc_conv_body(compute_deg, m_hbm, src_hbm, dst_hbm, *rest):
    if compute_deg:
        (p_hbm, deg_hbm, src_v, dst_v, rows_v, zb, ones_v, zline, acc, dacc,
         sem) = rest
    else:
        (p_hbm, src_v, dst_v, rows_v, zb, ones_v, zline, acc, dacc,
         sem) = rest
    c = lax.axis_index("c")
    s = lax.axis_index("s")
    w = c * NS + s

    zeros16 = jnp.zeros((16,), jnp.float32)
    ones16 = jnp.ones((16,), jnp.float32)
    for r in range(16):
        for cc in range(H // 16):
            zb[r, pl.ds(cc * 16, 16)] = zeros16
    for cc in range(CH // 16):
        ones_v[pl.ds(cc * 16, 16)] = ones16
        zline[pl.ds(cc * 16, 16)] = zeros16

    def zero_acc(i, carry):
        pltpu.sync_copy(zb, acc.at[pl.ds(s * ROWS_PER_TILE_Z + i * 16, 16)])
        return carry
    lax.fori_loop(0, ROWS_PER_TILE_Z // 16, zero_acc, 0)
    if compute_deg:
        def zero_deg(i, carry):
            pltpu.sync_copy(zline,
                            dacc.at[pl.ds(s * ROWS_PER_TILE_Z + i * CH, CH)])
            return carry
        lax.fori_loop(0, ROWS_PER_TILE_Z // CH, zero_deg, 0)
    plsc.subcore_barrier()

    pltpu.sync_copy(src_hbm.at[w], src_v)
    pltpu.sync_copy(dst_hbm.at[w], dst_v)

    def chunk(j, carry):
        idx_d = dst_v.at[j]
        pltpu.async_copy(m_hbm.at[src_v.at[j]], rows_v, sem).wait()
        pltpu.sync_copy(rows_v, acc.at[idx_d], add=True)
        if compute_deg:
            pltpu.sync_copy(ones_v, dacc.at[idx_d], add=True)
        return carry
    lax.fori_loop(0, NCHUNK, chunk, 0)
    plsc.subcore_barrier()

    base = s * ROWS_PER_TILE_Z
    pltpu.sync_copy(acc.at[pl.ds(base, ROWS_PER_TILE_Z)],
                    p_hbm.at[c, pl.ds(base, ROWS_PER_TILE_Z)])
    if compute_deg:
        @pl.when(s == 0)
        def _():
            pltpu.sync_copy(dacc, deg_hbm.at[pl.ds(c * NPAD, NPAD)])


def _sc_mesh(num_cores):
    return plsc.VectorSubcoreMesh(core_axis_name="c", subcore_axis_name="s",
                                  num_cores=num_cores, num_subcores=NS)


@functools.lru_cache(maxsize=None)
def _make_sc_conv(compute_deg):
    if compute_deg:
        out_type = (jax.ShapeDtypeStruct((NC, NPAD, H), jnp.float32),
                    jax.ShapeDtypeStruct((NC * NPAD,), jnp.float32))
    else:
        out_type = jax.ShapeDtypeStruct((NC, NPAD, H), jnp.float32)
    return pl.kernel(
        functools.partial(_sc_conv_body, compute_deg),
        name=f"sc_conv_{int(compute_deg)}",
        out_type=out_type,
        mesh=_sc_mesh(NC),
        scratch_types=[
            pltpu.VMEM((NCHUNK, CH), jnp.int32),
            pltpu.VMEM((NCHUNK, CH), jnp.int32),
            pltpu.VMEM((CH, H), jnp.float32),
            pltpu.VMEM((16, H), jnp.float32),
            pltpu.VMEM((CH,), jnp.float32),
            pltpu.VMEM((CH,), jnp.float32),
            pltpu.VMEM_SHARED((NPAD, H), jnp.float32),
            pltpu.VMEM_SHARED((NPAD,), jnp.float32),
            pltpu.SemaphoreType.DMA,
        ],
    )


def _mid_body(p_ref, degt_ref, bg_ref, wg_ref, m_ref):
    dsum = degt_ref[...]
    inv = jnp.where(dsum > 0.0, 1.0 / jnp.where(dsum > 0.0, dsum, 1.0), 0.0)
    out = jax.nn.relu((p_ref[0] + p_ref[1]) * inv + bg_ref[...])
    m_ref[...] = jnp.dot(out, wg_ref[...], preferred_element_type=jnp.float32)


def _run_mid_tc(p, degt, bg0, Wg1T):
    nb = N // BN_D
    full = lambda i: (0, 0)
    return pl.pallas_call(
        _mid_body,
        grid=(nb,),
        in_specs=[
            pl.BlockSpec((NC, BN_D, H), lambda i: (0, i, 0)),
            pl.BlockSpec((BN_D, 1), lambda i: (i, 0)),
            pl.BlockSpec((1, H), full),
            pl.BlockSpec((H, H), full),
        ],
        out_specs=pl.BlockSpec((BN_D, H), lambda i: (i, 0)),
        out_shape=jax.ShapeDtypeStruct((N, H), jnp.float32),
        compiler_params=pltpu.CompilerParams(
            dimension_semantics=("parallel",)),
    )(p, degt, bg0, Wg1T)


def _final_body(q_ref, degt_ref, h_ref, bg_ref, wskip_ref, bskip_ref,
                gamma_ref, beta_ref, wout_ref, bout_ref, y_ref):
    dsum = degt_ref[...]
    inv = jnp.where(dsum > 0.0, 1.0 / jnp.where(dsum > 0.0, dsum, 1.0), 0.0)
    out = jax.nn.relu((q_ref[0] + q_ref[1]) * inv + bg_ref[...])
    res = out + jnp.dot(h_ref[...], wskip_ref[...],
                        preferred_element_type=jnp.float32) + bskip_ref[...]
    mu = jnp.mean(res, axis=-1, keepdims=True)
    var = jnp.mean((res - mu) * (res - mu), axis=-1, keepdims=True)
    ln = gamma_ref[...] * (res - mu) / jnp.sqrt(var + 1e-5) + beta_ref[...]
    y_ref[...] = jnp.dot(ln, wout_ref[...],
                         preferred_element_type=jnp.float32) + bout_ref[...]


def _run_final_tc(q, degt, h, bg1, WskipT, bskip, gamma, beta, WoutT, bout):
    nb = N // BN_D
    full = lambda i: (0, 0)
    return pl.pallas_call(
        _final_body,
        grid=(nb,),
        in_specs=[
            pl.BlockSpec((NC, BN_D, H), lambda i: (0, i, 0)),
            pl.BlockSpec((BN_D, 1), lambda i: (i, 0)),
            pl.BlockSpec((BN_D, H), lambda i: (i, 0)),
            pl.BlockSpec((1, H), full),
            pl.BlockSpec((H, H), full),
            pl.BlockSpec((1, H), full),
            pl.BlockSpec((1, H), full),
            pl.BlockSpec((1, H), full),
            pl.BlockSpec((H, OUT), full),
            pl.BlockSpec((1, OUT), full),
        ],
        out_specs=pl.BlockSpec((BN_D, OUT), lambda i: (i, 0)),
        out_shape=jax.ShapeDtypeStruct((N, OUT), jnp.float32),
        compiler_params=pltpu.CompilerParams(
            dimension_semantics=("parallel",)),
    )(q, degt, h, bg1, WskipT, bskip, gamma, beta, WoutT, bout)


def kernel(x, edge_index, Wih0, Whh0, bih0, bhh0, Wih1, Whh1, bih1, bhh1,
           Wg0, bg0, Wg1, bg1, Wskip, bskip, gamma, beta, Wout, bout):
    src = edge_index[0]
    dst = edge_index[1]
    pad = EPAD - E

    srcp = jnp.concatenate([src, jnp.zeros((pad,), jnp.int32)]
                           ).reshape(NW, NCHUNK, CH)
    dstp = jnp.concatenate([dst, jnp.full((pad,), N, jnp.int32)]
                           ).reshape(NW, NCHUNK, CH)

    x2 = x.reshape(N, T * H)

    row = lambda v: v.reshape(1, -1)
    h, m0 = _run_gru_tc(x2, Wih0.T, Whh0.T, row(bih0), row(bhh0),
                        Wih1.T, Whh1.T, row(bih1), row(bhh1), Wg0.T)

    p, deg_flat = _make_sc_conv(True)(m0, srcp, dstp)
    degt = (deg_flat[:N] + deg_flat[NPAD:NPAD + N]).reshape(N, 1)

    m1 = _run_mid_tc(p, degt, row(bg0), Wg1.T)
    q = _make_sc_conv(False)(m1, srcp, dstp)

    y = _run_final_tc(q, degt, h, row(bg1), Wskip.T, row(bskip),
                      row(gamma), row(beta), Wout.T, row(bout))
    return y
